# Initial kernel scaffold; baseline (speedup 1.0000x reference)
import jax
import jax.numpy as jnp
from jax.experimental import pallas as pl


def kernel(x, lengths_in, lengths_aureg, mask_aureg, W_ih0, W_hh0, b_ih0, b_hh0, W_ih1, W_hh1, b_ih1, b_hh1):
    raise NotImplementedError("write your pallas kernel here")


# speedup vs baseline: 7.7208x; 7.7208x over previous
import jax
import jax.numpy as jnp
from jax.experimental import pallas as pl

B = 16
T = 512
D = 256
H = 256
TA = 64


def _cell(g, c):
    i = jax.nn.sigmoid(g[:, 0 * H:1 * H])
    f = jax.nn.sigmoid(g[:, 1 * H:2 * H])
    gg = jnp.tanh(g[:, 2 * H:3 * H])
    o = jax.nn.sigmoid(g[:, 3 * H:4 * H])
    c2 = f * c + i * gg
    h2 = o * jnp.tanh(c2)
    return h2, c2


def _lstm_kernel(xT_ref, lin_ref, lar_ref, w0T_ref, w1T_ref, b0_ref, b1_ref,
                 teafo_ref, aureg_ref):
    f32 = jnp.float32
    zero = jnp.zeros((B, H), dtype=f32)

    def tf_step(t, carry):
        h0, c0, h1, c1 = carry
        x_t = xT_ref[t]
        g0 = jnp.dot(jnp.concatenate([x_t, h0], axis=1), w0T_ref[:],
                     preferred_element_type=f32) + b0_ref[:]
        h0n, c0n = _cell(g0, c0)
        g1 = jnp.dot(jnp.concatenate([h0n, h1], axis=1), w1T_ref[:],
                     preferred_element_type=f32) + b1_ref[:]
        h1n, c1n = _cell(g1, c1)
        m = lin_ref[:] > t
        teafo_ref[t] = jnp.where(m, h1n, 0.0)
        h0 = jnp.where(m, h0n, h0)
        c0 = jnp.where(m, c0n, c0)
        h1 = jnp.where(m, h1n, h1)
        c1 = jnp.where(m, c1n, c1)
        return h0, c0, h1, c1

    h0, c0, h1, c1 = jax.lax.fori_loop(
        0, T, tf_step, (zero, zero, zero, zero), unroll=False)

    def ar_step(t, carry):
        h0, c0, h1, c1, inp = carry
        g0 = jnp.dot(jnp.concatenate([inp, h0], axis=1), w0T_ref[:],
                     preferred_element_type=f32) + b0_ref[:]
        h0n, c0n = _cell(g0, c0)
        g1 = jnp.dot(jnp.concatenate([h0n, h1], axis=1), w1T_ref[:],
                     preferred_element_type=f32) + b1_ref[:]
        h1n, c1n = _cell(g1, c1)
        m = lar_ref[:] > t
        out = jnp.where(m, h1n, 0.0)
        aureg_ref[t] = out
        h0 = jnp.where(m, h0n, h0)
        c0 = jnp.where(m, c0n, c0)
        h1 = jnp.where(m, h1n, h1)
        c1 = jnp.where(m, c1n, c1)
        return h0, c0, h1, c1, out

    jax.lax.fori_loop(0, TA, ar_step, (h0, c0, h1, c1, h1), unroll=False)


def kernel(x, lengths_in, lengths_aureg, mask_aureg, W_ih0, W_hh0, b_ih0,
           b_hh0, W_ih1, W_hh1, b_ih1, b_hh1):
    f32 = jnp.float32
    xT = jnp.transpose(x, (1, 0, 2))
    w0T = jnp.concatenate([W_ih0, W_hh0], axis=1).T
    w1T = jnp.concatenate([W_ih1, W_hh1], axis=1).T
    b0 = (b_ih0 + b_hh0).reshape(1, 4 * H)
    b1 = (b_ih1 + b_hh1).reshape(1, 4 * H)
    lin = jnp.broadcast_to(lengths_in[:, None], (B, H))
    lar = jnp.broadcast_to(lengths_aureg[:, None], (B, H))

    teafo_raw, aureg_raw = pl.pallas_call(
        _lstm_kernel,
        out_shape=(
            jax.ShapeDtypeStruct((T, B, H), f32),
            jax.ShapeDtypeStruct((TA, B, H), f32),
        ),
    )(xT, lin, lar, w0T, w1T, b0, b1)

    teafo = jnp.transpose(teafo_raw, (1, 0, 2))
    aureg = jnp.transpose(aureg_raw, (1, 0, 2))
    return (teafo, aureg)
